# baseline (speedup 1.0000x reference)
import jax
import jax.numpy as jnp
from jax.experimental import pallas as pl
from jax.experimental.pallas import tpu as pltpu

DIM = 1024
HID = 2048
E = 16
B = 128
LOSS_COEF = 0.1


def _fused_kernel(te_ref, x_ref, wg0_ref, bg0_ref, wg1_ref, bg1_ref, wg2_ref,
                  bg2_ref, w1_ref, b1_ref, w2_ref, b2_ref,
                  loss_ref, out_ref, chosen_ref):
    s = pl.program_id(0)

    @pl.when(s == 0)
    def _gate():
        x = x_ref[...]
        h = jnp.dot(x, wg0_ref[...], preferred_element_type=jnp.float32) + bg0_ref[...]
        h = jnp.dot(h, wg1_ref[...], preferred_element_type=jnp.float32) + bg1_ref[...]
        preds = jnp.dot(h, wg2_ref[...], preferred_element_type=jnp.float32) + bg2_ref[...]

        m = jnp.max(preds, axis=1, keepdims=True)
        logz = m + jnp.log(jnp.sum(jnp.exp(preds - m), axis=1, keepdims=True))
        iota = jax.lax.broadcasted_iota(jnp.int32, (B, E), 1)
        te = te_ref[...]
        picked = jnp.sum(jnp.where(iota == te, preds, 0.0), axis=1, keepdims=True)
        loss_ref[...] = jnp.sum(logz - picked, axis=0, keepdims=True) * (LOSS_COEF / B)

        chosen_ref[...] = jnp.min(jnp.where(preds == m, iota, E), axis=1, keepdims=True)
        out_ref[...] = jnp.zeros_like(out_ref)

    @pl.when(s > 0)
    def _expert():
        e = s - 1
        h = jnp.dot(x_ref[...], w1_ref[0], preferred_element_type=jnp.float32) + b1_ref[0]
        h = jnp.maximum(h, 0.0)
        oe = jnp.dot(h, w2_ref[0], preferred_element_type=jnp.float32) + b2_ref[0]
        mask = chosen_ref[...] == e
        out_ref[...] += jnp.where(mask, oe, 0.0)


def kernel(inputs, true_experts, Wg0, bg0, Wg1, bg1, Wg2, bg2, W1, b1, W2, b2):
    te = true_experts.astype(jnp.int32).reshape(B, 1)
    x_flat = inputs.reshape(B, inputs.shape[1] * DIM)

    def _e(s):
        return jnp.maximum(s - 1, 0)

    loss2d, out = pl.pallas_call(
        _fused_kernel,
        grid=(E + 1,),
        in_specs=[
            pl.BlockSpec((B, 1), lambda s: (0, 0)),
            pl.BlockSpec((B, DIM), lambda s: (0, 0)),
            pl.BlockSpec((DIM, HID), lambda s: (0, 0)),
            pl.BlockSpec((1, HID), lambda s: (0, 0)),
            pl.BlockSpec((HID, HID), lambda s: (0, 0)),
            pl.BlockSpec((1, HID), lambda s: (0, 0)),
            pl.BlockSpec((HID, E), lambda s: (0, 0)),
            pl.BlockSpec((1, E), lambda s: (0, 0)),
            pl.BlockSpec((1, DIM, HID), lambda s: (_e(s), 0, 0)),
            pl.BlockSpec((1, 1, HID), lambda s: (_e(s), 0, 0)),
            pl.BlockSpec((1, HID, DIM), lambda s: (_e(s), 0, 0)),
            pl.BlockSpec((1, 1, DIM), lambda s: (_e(s), 0, 0)),
        ],
        out_specs=(
            pl.BlockSpec((1, 1), lambda s: (0, 0)),
            pl.BlockSpec((B, DIM), lambda s: (0, 0)),
        ),
        out_shape=(
            jax.ShapeDtypeStruct((1, 1), jnp.float32),
            jax.ShapeDtypeStruct((B, DIM), jnp.float32),
        ),
        scratch_shapes=[
            pltpu.VMEM((B, 1), jnp.int32),
        ],
        compiler_params=pltpu.CompilerParams(
            vmem_limit_bytes=64 * 1024 * 1024,
        ),
    )(te, x_flat, Wg0, bg0.reshape(1, HID), Wg1, bg1.reshape(1, HID),
      Wg2, bg2.reshape(1, E), W1, b1.reshape(E, 1, HID), W2,
      b2.reshape(E, 1, DIM))

    return (out, loss2d[0, 0])


# speedup vs baseline: 1.0432x; 1.0432x over previous
import jax
import jax.numpy as jnp
from jax.experimental import pallas as pl
from jax.experimental.pallas import tpu as pltpu

DIM = 1024
HID = 2048
E = 16
B = 128
LOSS_COEF = 0.1


def _fused_kernel(te_ref, x_ref, wg0_ref, bg0_ref, wg1_ref, bg1_ref, wg2_ref,
                  bg2_ref, w1_ref, b1_ref, w2_ref, b2_ref,
                  loss_ref, out_ref, chosen_ref, xs_ref):
    s = pl.program_id(0)

    @pl.when(s == 0)
    def _gate():
        x = x_ref[:, 0, :]
        xs_ref[...] = x
        h = jnp.dot(x, wg0_ref[...], preferred_element_type=jnp.float32) + bg0_ref[...]
        h = jnp.dot(h, wg1_ref[...], preferred_element_type=jnp.float32) + bg1_ref[...]
        preds = jnp.dot(h, wg2_ref[...], preferred_element_type=jnp.float32) + bg2_ref[...]

        m = jnp.max(preds, axis=1, keepdims=True)
        logz = m + jnp.log(jnp.sum(jnp.exp(preds - m), axis=1, keepdims=True))
        iota = jax.lax.broadcasted_iota(jnp.int32, (B, E), 1)
        te = te_ref[...]
        picked = jnp.sum(jnp.where(iota == te, preds, 0.0), axis=1, keepdims=True)
        loss_ref[...] = jnp.sum(logz - picked, axis=0, keepdims=True) * (LOSS_COEF / B)

        chosen_ref[...] = jnp.min(jnp.where(preds == m, iota, E), axis=1, keepdims=True)
        out_ref[...] = jnp.zeros_like(out_ref)

    @pl.when(s > 0)
    def _expert():
        e = s - 1
        h = jnp.dot(xs_ref[...], w1_ref[0], preferred_element_type=jnp.float32) + b1_ref[0]
        h = jnp.maximum(h, 0.0)
        oe = jnp.dot(h, w2_ref[0], preferred_element_type=jnp.float32) + b2_ref[0]
        mask = chosen_ref[...] == e
        out_ref[...] += jnp.where(mask, oe, 0.0)


def kernel(inputs, true_experts, Wg0, bg0, Wg1, bg1, Wg2, bg2, W1, b1, W2, b2):
    te = true_experts.astype(jnp.int32).reshape(B, 1)
    n_tok = inputs.shape[1]

    def _e(s):
        return jnp.maximum(s - 1, 0)

    loss2d, out = pl.pallas_call(
        _fused_kernel,
        grid=(E + 1,),
        in_specs=[
            pl.BlockSpec((B, 1), lambda s: (0, 0)),
            pl.BlockSpec((B, n_tok, DIM), lambda s: (0, 0, 0)),
            pl.BlockSpec((DIM, HID), lambda s: (0, 0)),
            pl.BlockSpec((1, HID), lambda s: (0, 0)),
            pl.BlockSpec((HID, HID), lambda s: (0, 0)),
            pl.BlockSpec((1, HID), lambda s: (0, 0)),
            pl.BlockSpec((HID, E), lambda s: (0, 0)),
            pl.BlockSpec((1, E), lambda s: (0, 0)),
            pl.BlockSpec((1, DIM, HID), lambda s: (_e(s), 0, 0)),
            pl.BlockSpec((1, 1, HID), lambda s: (_e(s), 0, 0)),
            pl.BlockSpec((1, HID, DIM), lambda s: (_e(s), 0, 0)),
            pl.BlockSpec((1, 1, DIM), lambda s: (_e(s), 0, 0)),
        ],
        out_specs=(
            pl.BlockSpec((1, 1), lambda s: (0, 0)),
            pl.BlockSpec((B, DIM), lambda s: (0, 0)),
        ),
        out_shape=(
            jax.ShapeDtypeStruct((1, 1), jnp.float32),
            jax.ShapeDtypeStruct((B, DIM), jnp.float32),
        ),
        scratch_shapes=[
            pltpu.VMEM((B, 1), jnp.int32),
            pltpu.VMEM((B, DIM), jnp.float32),
        ],
        compiler_params=pltpu.CompilerParams(
            vmem_limit_bytes=64 * 1024 * 1024,
        ),
    )(te, inputs, Wg0, bg0.reshape(1, HID), Wg1, bg1.reshape(1, HID),
      Wg2, bg2.reshape(1, E), W1, b1.reshape(E, 1, HID), W2,
      b2.reshape(E, 1, DIM))

    return (out, loss2d[0, 0])
